# baseline (speedup 1.0000x reference)
import functools

import jax
import jax.numpy as jnp
from jax import lax
from jax.experimental import pallas as pl
from jax.experimental.pallas import tpu as pltpu
from jax.experimental.pallas import tpu_sc as plsc
from jax._src.pallas import mpmd as _mpmd

NB = 4096
HID = 256
NLIN = NB * NB
NC = 2
NS = 16
NW = NC * NS
CW = 128
CH = 42
NQ = 6
RQ = CH // NQ
PT = CH * CW
NP = NW * PT


def _lanes():
    return lax.iota(jnp.int32, 16)

_mesh = plsc.VectorSubcoreMesh(core_axis_name="c", subcore_axis_name="s")


def _mlp_body(ppm, wi, bi, wo, bo, wd, bd, to_out, td_out):
    dn = (((1,), (1,)), ((), ()))
    x = jnp.tanh(
        lax.dot_general(ppm[...], wi[...], dn,
                        preferred_element_type=jnp.float32,
                        precision=lax.Precision.HIGHEST) + bi[...])
    to_out[...] = jnp.tanh(
        lax.dot_general(x, wo[...], dn,
                        preferred_element_type=jnp.float32,
                        precision=lax.Precision.HIGHEST) + bo[...])
    td_out[...] = jnp.tanh(
        lax.dot_general(x, wd[...], dn,
                        preferred_element_type=jnp.float32,
                        precision=lax.Precision.HIGHEST) + bd[...])


_mlp = pl.pallas_call(
    _mlp_body,
    out_shape=(jax.ShapeDtypeStruct((1, NB), jnp.float32),
               jax.ShapeDtypeStruct((1, NB), jnp.float32)),
    compiler_params=pltpu.CompilerParams(vmem_limit_bytes=100 * 1024 * 1024),
)


def _zeros_body(o_ref):
    o_ref[...] = jnp.zeros_like(o_ref)


_zeros = pl.pallas_call(
    _zeros_body,
    out_shape=jax.ShapeDtypeStruct((NLIN,), jnp.float32),
    grid=(16,),
    out_specs=pl.BlockSpec((NLIN // 16,), lambda i: (i,)),
)


def _wid():
    return lax.axis_index("s") * NC + lax.axis_index("c")


def _fire_drain(mk):
    for q in range(NQ):
        def fire(i, c, q=q):
            mk(q * RQ + i, q).start()
            return c
        lax.fori_loop(0, RQ, fire, 0)
    for q in range(NQ):
        def drain(i, c, q=q):
            mk(q * RQ + i, q).wait()
            return c
        lax.fori_loop(0, RQ, drain, 0)


@functools.partial(
    pl.kernel,
    out_type=(jax.ShapeDtypeStruct((NW, CH, CW), jnp.int32),
              jax.ShapeDtypeStruct((NW, CH, CW), jnp.float32),
              jax.ShapeDtypeStruct((NW, 16), jnp.float32),
              jax.ShapeDtypeStruct((NLIN,), jnp.int32)),
    mesh=_mesh,
    compiler_params=pltpu.CompilerParams(needs_layout_passes=False),
    scratch_types=[pltpu.VMEM((CH, CW), jnp.int32),
                   pltpu.VMEM((CH, CW), jnp.int32),
                   pltpu.VMEM((NB,), jnp.float32),
                   pltpu.VMEM((NB,), jnp.float32),
                   pltpu.VMEM((CH, CW), jnp.int32),
                   pltpu.VMEM((CH, CW), jnp.float32),
                   pltpu.VMEM((CH, CW), jnp.int32),
                   pltpu.VMEM((16,), jnp.float32),
                   pltpu.SemaphoreType.DMA((NQ,))],
)
def _prep(a0_hbm, a1_hbm, to_hbm, td_hbm,
          lin_hbm, val_hbm, tmax_hbm, table_hbm,
          a0v, a1v, tov, tdv, linv, valv, idsv, mx16, sem):
    w = _wid()
    pltpu.sync_copy(a0_hbm.at[w], a0v)
    pltpu.sync_copy(a1_hbm.at[w], a1v)
    pltpu.sync_copy(to_hbm, tov)
    pltpu.sync_copy(td_hbm, tdv)
    base = w * PT

    def chunk(j, m):
        for k in range(CW // 16):
            off = k * 16
            av = a0v[j, pl.ds(off, 16)]
            bv = a1v[j, pl.ds(off, 16)]
            linv[j, pl.ds(off, 16)] = av * NB + bv
            idsv[j, pl.ds(off, 16)] = base + j * CW + off + _lanes()
            v = plsc.load_gather(tov, [av]) * plsc.load_gather(tdv, [bv])
            valv[j, pl.ds(off, 16)] = v
            m = jnp.maximum(m, v)
        return m

    m16 = lax.fori_loop(0, CH, chunk, jnp.full((16,), -2.0, jnp.float32))
    mx16[...] = jnp.broadcast_to(jnp.max(m16), (16,))
    pltpu.sync_copy(mx16, tmax_hbm.at[w])
    pltpu.sync_copy(linv, lin_hbm.at[w])
    pltpu.sync_copy(valv, val_hbm.at[w])

    _fire_drain(lambda j, q: pltpu.make_async_copy(
        idsv.at[j], table_hbm.at[linv.at[j]], sem.at[q]))


@functools.partial(
    pl.kernel,
    out_type=jax.ShapeDtypeStruct((NW, 16), jnp.float32),
    mesh=_mesh,
    compiler_params=pltpu.CompilerParams(needs_layout_passes=False),
    scratch_types=[pltpu.VMEM((CH, CW), jnp.int32),
                   pltpu.VMEM((CH, CW), jnp.float32),
                   pltpu.VMEM((CH, CW), jnp.int32),
                   pltpu.VMEM((NW, 16), jnp.float32),
                   pltpu.VMEM((16,), jnp.float32),
                   pltpu.SemaphoreType.DMA((NQ,))],
)
def _denom(lin_hbm, val_hbm, table_hbm, tmax_hbm, dnm_hbm,
           linv, valv, wv, tmv, st16, sem):
    w = _wid()
    pltpu.sync_copy(lin_hbm.at[w], linv)
    pltpu.sync_copy(val_hbm.at[w], valv)
    pltpu.sync_copy(tmax_hbm, tmv)

    m16 = lax.fori_loop(0, NW, lambda r, m: jnp.maximum(m, tmv[r]),
                        jnp.full((16,), -2.0, jnp.float32))

    _fire_drain(lambda j, q: pltpu.make_async_copy(
        table_hbm.at[linv.at[j]], wv.at[j], sem.at[q]))
    base = w * PT

    def chunk(j, acc):
        for k in range(CW // 16):
            off = k * 16
            ids = base + j * CW + off + _lanes()
            e = jnp.exp(valv[j, pl.ds(off, 16)] - m16)
            acc = acc + jnp.where(wv[j, pl.ds(off, 16)] == ids, e, 0.0)
        return acc

    s16 = lax.fori_loop(0, CH, chunk, jnp.zeros((16,), jnp.float32))
    st16[...] = jnp.broadcast_to(jnp.sum(s16), (16,))
    pltpu.sync_copy(st16, dnm_hbm.at[w])


def _scatter_body(lin_hbm, val_hbm, tmax_hbm, dnm_hbm, zin_hbm, out_hbm,
                  linv, valv, evv, tmv, dmv, sem):
    del zin_hbm
    w = _wid()
    pltpu.sync_copy(lin_hbm.at[w], linv)
    pltpu.sync_copy(val_hbm.at[w], valv)
    pltpu.sync_copy(tmax_hbm, tmv)
    pltpu.sync_copy(dnm_hbm, dmv)

    m16 = lax.fori_loop(0, NW, lambda r, mm: jnp.maximum(mm, tmv[r]),
                        jnp.full((16,), -2.0, jnp.float32))
    d16 = lax.fori_loop(0, NW, lambda r, ss: ss + dmv[r],
                        jnp.zeros((16,), jnp.float32))
    inv16 = jnp.full((16,), 1.0, jnp.float32) / d16

    def chunk(j, c):
        for k in range(CW // 16):
            off = k * 16
            evv[j, pl.ds(off, 16)] = (
                jnp.exp(valv[j, pl.ds(off, 16)] - m16) * inv16)
        return c

    lax.fori_loop(0, CH, chunk, 0)

    _fire_drain(lambda j, q: pltpu.make_async_copy(
        evv.at[j], out_hbm.at[linv.at[j]], sem.at[q]))


_scatter = _mpmd._mpmd_map(
    [(_mesh, _scatter_body)],
    jax.ShapeDtypeStruct((NLIN,), jnp.float32),
    input_output_aliases={4: 0},
    compiler_params=pltpu.CompilerParams(needs_layout_passes=False),
    scratch_types=[pltpu.VMEM((CH, CW), jnp.int32),
                   pltpu.VMEM((CH, CW), jnp.float32),
                   pltpu.VMEM((CH, CW), jnp.float32),
                   pltpu.VMEM((NW, 16), jnp.float32),
                   pltpu.VMEM((NW, 16), jnp.float32),
                   pltpu.SemaphoreType.DMA((NQ,))],
)


def kernel(possible_actions, player_presence_map, W_in, b_in, W_to, b_to,
           W_td, b_td):
    a0 = possible_actions[:, 0].astype(jnp.int32)
    a1 = possible_actions[:, 1].astype(jnp.int32)
    pad = NP - a0.shape[0]
    a0p = jnp.concatenate([a0, jnp.broadcast_to(a0[:1], (pad,))])
    a1p = jnp.concatenate([a1, jnp.broadcast_to(a1[:1], (pad,))])
    a0p = a0p.reshape(NW, CH, CW)
    a1p = a1p.reshape(NW, CH, CW)

    to2, td2 = _mlp(player_presence_map.reshape(1, NB), W_in,
                    b_in.reshape(1, HID), W_to, b_to.reshape(1, NB),
                    W_td, b_td.reshape(1, NB))
    to = to2.reshape(NB)
    td = td2.reshape(NB)

    zout = _zeros()
    lin, val, tmax, table = _prep(a0p, a1p, to, td)
    dnm = _denom(lin, val, table, tmax)
    out = _scatter(lin, val, tmax, dnm, zout)
    return out.reshape(1, NLIN)


# speedup vs baseline: 3.7779x; 3.7779x over previous
import functools

import jax
import jax.numpy as jnp
from jax import lax
from jax.experimental import pallas as pl
from jax.experimental.pallas import tpu as pltpu
from jax.experimental.pallas import tpu_sc as plsc
from jax._src.pallas import mpmd as _mpmd

NB = 4096
HID = 256
NLIN = NB * NB
NC = 2
NS = 16
NW = NC * NS
PT = 5248
NP = NW * PT
GB = 16


def _lanes():
    return lax.iota(jnp.int32, 16)

_mesh = plsc.VectorSubcoreMesh(core_axis_name="c", subcore_axis_name="s")


def _mlp_body(ppm, wi, bi, wo, bo, wd, bd, to_out, td_out):
    dn = (((1,), (1,)), ((), ()))
    x = jnp.tanh(
        lax.dot_general(ppm[...], wi[...], dn,
                        preferred_element_type=jnp.float32,
                        precision=lax.Precision.HIGHEST) + bi[...])
    to_out[...] = jnp.tanh(
        lax.dot_general(x, wo[...], dn,
                        preferred_element_type=jnp.float32,
                        precision=lax.Precision.HIGHEST) + bo[...])
    td_out[...] = jnp.tanh(
        lax.dot_general(x, wd[...], dn,
                        preferred_element_type=jnp.float32,
                        precision=lax.Precision.HIGHEST) + bd[...])


_mlp = pl.pallas_call(
    _mlp_body,
    out_shape=(jax.ShapeDtypeStruct((1, NB), jnp.float32),
               jax.ShapeDtypeStruct((1, NB), jnp.float32)),
    compiler_params=pltpu.CompilerParams(vmem_limit_bytes=100 * 1024 * 1024),
)


def _zeros_body(o_ref):
    o_ref[...] = jnp.zeros_like(o_ref)


_zeros = pl.pallas_call(
    _zeros_body,
    out_shape=jax.ShapeDtypeStruct((NLIN,), jnp.float32),
    grid=(GB,),
    out_specs=pl.BlockSpec((NLIN // GB,), lambda i: (i,)),
)


def _sum_body(x_ref, o_ref):
    s = jnp.full((1, 1), jnp.sum(x_ref[...]), jnp.float32)
    o_ref[...] = jnp.where(pl.program_id(0) == 0, s, o_ref[...] + s)


_sumk = pl.pallas_call(
    _sum_body,
    out_shape=jax.ShapeDtypeStruct((1, 1), jnp.float32),
    grid=(GB,),
    in_specs=[pl.BlockSpec((NLIN // GB,), lambda i: (i,))],
    out_specs=pl.BlockSpec((1, 1), lambda i: (0, 0)),
)


def _scale_body(x_ref, d_ref, o_ref):
    o_ref[...] = x_ref[...] * (1.0 / d_ref[0, 0])


_scale = pl.pallas_call(
    _scale_body,
    out_shape=jax.ShapeDtypeStruct((NLIN,), jnp.float32),
    grid=(GB,),
    in_specs=[pl.BlockSpec((NLIN // GB,), lambda i: (i,)),
              pl.BlockSpec((1, 1), lambda i: (0, 0))],
    out_specs=pl.BlockSpec((NLIN // GB,), lambda i: (i,)),
    input_output_aliases={0: 0},
)


def _wid():
    return lax.axis_index("s") * NC + lax.axis_index("c")


@functools.partial(
    pl.kernel,
    out_type=(jax.ShapeDtypeStruct((NW, PT), jnp.int32),
              jax.ShapeDtypeStruct((NW, PT), jnp.float32),
              jax.ShapeDtypeStruct((NW, 16), jnp.float32)),
    mesh=_mesh,
    compiler_params=pltpu.CompilerParams(needs_layout_passes=False),
    scratch_types=[pltpu.VMEM((PT,), jnp.int32),
                   pltpu.VMEM((PT,), jnp.int32),
                   pltpu.VMEM((NB,), jnp.float32),
                   pltpu.VMEM((NB,), jnp.float32),
                   pltpu.VMEM((PT,), jnp.int32),
                   pltpu.VMEM((PT,), jnp.float32),
                   pltpu.VMEM((16,), jnp.float32)],
)
def _prep(a0_hbm, a1_hbm, to_hbm, td_hbm,
          lin_hbm, val_hbm, tmax_hbm,
          a0v, a1v, tov, tdv, linv, valv, mx16):
    w = _wid()
    pltpu.sync_copy(a0_hbm.at[w], a0v)
    pltpu.sync_copy(a1_hbm.at[w], a1v)
    pltpu.sync_copy(to_hbm, tov)
    pltpu.sync_copy(td_hbm, tdv)

    def chunk(i, m):
        off = i * 16
        av = a0v[pl.ds(off, 16)]
        bv = a1v[pl.ds(off, 16)]
        linv[pl.ds(off, 16)] = av * NB + bv
        v = plsc.load_gather(tov, [av]) * plsc.load_gather(tdv, [bv])
        valv[pl.ds(off, 16)] = v
        return jnp.maximum(m, v)

    m16 = lax.fori_loop(0, PT // 16, chunk, jnp.full((16,), -2.0, jnp.float32))
    mx16[...] = jnp.broadcast_to(jnp.max(m16), (16,))
    pltpu.sync_copy(mx16, tmax_hbm.at[w])
    pltpu.sync_copy(linv, lin_hbm.at[w])
    pltpu.sync_copy(valv, val_hbm.at[w])


def _scatter_body(lin_hbm, val_hbm, tmax_hbm, zin_hbm, out_hbm,
                  linv, valv, evv, tmv, sem):
    del zin_hbm
    w = _wid()
    pltpu.sync_copy(lin_hbm.at[w], linv)
    pltpu.sync_copy(val_hbm.at[w], valv)
    pltpu.sync_copy(tmax_hbm, tmv)

    m16 = lax.fori_loop(0, NW, lambda r, mm: jnp.maximum(mm, tmv[r]),
                        jnp.full((16,), -2.0, jnp.float32))

    def chunk(i, c):
        off = i * 16
        evv[pl.ds(off, 16)] = jnp.exp(valv[pl.ds(off, 16)] - m16)
        return c

    lax.fori_loop(0, PT // 16, chunk, 0)

    pltpu.async_copy(evv, out_hbm.at[linv], sem).wait()


_scatter = _mpmd._mpmd_map(
    [(_mesh, _scatter_body)],
    jax.ShapeDtypeStruct((NLIN,), jnp.float32),
    input_output_aliases={3: 0},
    compiler_params=pltpu.CompilerParams(needs_layout_passes=False),
    scratch_types=[pltpu.VMEM((PT,), jnp.int32),
                   pltpu.VMEM((PT,), jnp.float32),
                   pltpu.VMEM((PT,), jnp.float32),
                   pltpu.VMEM((NW, 16), jnp.float32),
                   pltpu.SemaphoreType.DMA],
)


def kernel(possible_actions, player_presence_map, W_in, b_in, W_to, b_to,
           W_td, b_td):
    a0 = possible_actions[:, 0].astype(jnp.int32)
    a1 = possible_actions[:, 1].astype(jnp.int32)
    pad = NP - a0.shape[0]
    a0p = jnp.concatenate([a0, jnp.broadcast_to(a0[:1], (pad,))])
    a1p = jnp.concatenate([a1, jnp.broadcast_to(a1[:1], (pad,))])
    a0p = a0p.reshape(NW, PT)
    a1p = a1p.reshape(NW, PT)

    to2, td2 = _mlp(player_presence_map.reshape(1, NB), W_in,
                    b_in.reshape(1, HID), W_to, b_to.reshape(1, NB),
                    W_td, b_td.reshape(1, NB))
    to = to2.reshape(NB)
    td = td2.reshape(NB)

    zout = _zeros()
    lin, val, tmax = _prep(a0p, a1p, to, td)
    unnorm = _scatter(lin, val, tmax, zout)
    denom = _sumk(unnorm)
    out = _scale(unnorm, denom)
    return out.reshape(1, NLIN)
